# Initial kernel scaffold; baseline (speedup 1.0000x reference)
import jax
import jax.numpy as jnp
from jax.experimental import pallas as pl


def kernel(output, target):
    raise NotImplementedError("write your pallas kernel here")


# speedup vs baseline: 2.3730x; 2.3730x over previous
import jax
import jax.numpy as jnp
from jax.experimental import pallas as pl
from jax.experimental.pallas import tpu as pltpu

_N = 16384
_C = 1000
_BM = 256
_GRID = _N // _BM
_THRESHOLD = 0.5
_EPS = 1e-08


def _body(tgt_ref, x_ref, out_ref, tp_acc, p_acc, cnt_acc, m_acc):
    step = pl.program_id(0)

    @pl.when(step == 0)
    def _init():
        tp_acc[...] = jnp.zeros_like(tp_acc)
        p_acc[...] = jnp.zeros_like(p_acc)
        cnt_acc[...] = jnp.zeros_like(cnt_acc)
        m_acc[0] = 0.0

    x = x_ref[...]
    sig = 1.0 / (1.0 + jnp.exp(-x))
    tgt = tgt_ref[0, 0, :]
    col = jax.lax.broadcasted_iota(jnp.int32, (_BM, _C), 1)
    oh = (col == tgt[:, None]).astype(jnp.float32)
    pred = (sig >= _THRESHOLD).astype(jnp.float32)

    p_acc[...] += jnp.sum(pred, axis=0)
    tp_acc[...] += jnp.sum(pred * oh, axis=0)
    cnt_acc[...] += jnp.sum(oh, axis=0)

    mism = jnp.sum((sig != oh).astype(jnp.float32), axis=1)
    m_acc[0] += jnp.sum((mism == 0.0).astype(jnp.float32))

    @pl.when(step == _GRID - 1)
    def _fin():
        tp_raw = tp_acc[...]
        p = p_acc[...]
        cnt = cnt_acc[...]
        tp = tp_raw + _EPS
        fp = (p - tp_raw) + _EPS
        fn = (cnt - tp_raw) + _EPS
        tn = (_N - p - cnt + tp_raw) + _EPS
        precision = tp / (tp + fp + _EPS)
        recall = tp / (tp + fn + _EPS)
        f1 = 2.0 * precision * recall / (precision + recall + _EPS)

        zero_one = m_acc[0] / _N
        tp_s = jnp.sum(tp)
        tn_s = jnp.sum(tn)
        fp_s = jnp.sum(fp)
        fn_s = jnp.sum(fn)
        accuracy = (tp_s + tn_s) / (tp_s + tn_s + fp_s + fn_s)
        precision_g = tp_s / (tp_s + fp_s + _EPS)
        recall_g = tp_s / (tp_s + fn_s + _EPS)
        f1_g = 2.0 * precision_g * recall_g / (precision_g + recall_g + _EPS)
        precision_pc = jnp.sum(precision) / _C
        recall_pc = jnp.sum(recall) / _C
        f1_pc = jnp.sum(f1) / _C

        ones = jnp.ones((1, 128), jnp.float32)
        out_ref[0:1, :] = ones * zero_one
        out_ref[1:2, :] = ones * accuracy
        out_ref[2:3, :] = ones * precision_g
        out_ref[3:4, :] = ones * recall_g
        out_ref[4:5, :] = ones * f1_g
        out_ref[5:6, :] = ones * precision_pc
        out_ref[6:7, :] = ones * recall_pc
        out_ref[7:8, :] = ones * f1_pc


def kernel(output, target):
    tgt3 = target.reshape(_GRID, 1, _BM)
    out = pl.pallas_call(
        _body,
        grid=(_GRID,),
        in_specs=[
            pl.BlockSpec((1, 1, _BM), lambda i: (i, 0, 0)),
            pl.BlockSpec((_BM, _C), lambda i: (i, 0)),
        ],
        out_specs=pl.BlockSpec((8, 128), lambda i: (0, 0)),
        out_shape=jax.ShapeDtypeStruct((8, 128), jnp.float32),
        scratch_shapes=[
            pltpu.VMEM((_C,), jnp.float32),
            pltpu.VMEM((_C,), jnp.float32),
            pltpu.VMEM((_C,), jnp.float32),
            pltpu.SMEM((1,), jnp.float32),
        ],
        compiler_params=pltpu.CompilerParams(
            dimension_semantics=("arbitrary",)),
    )(tgt3, output)
    return tuple(out[i, 0] for i in range(8))
